# baseline (speedup 1.0000x reference)
import functools

import jax
import jax.numpy as jnp
from jax import lax
from jax.experimental import pallas as pl
from jax.experimental.pallas import tpu as pltpu
from jax.experimental.pallas import tpu_sc as plsc

Q = 512
D = 768
N = 100000
BN = 2048
NB = (N + BN - 1) // BN
NPAD = NB * BN
G = 128
NGB = BN // G
NGRP = NB * NGB
NSEL = 24
NCAND = NSEL * G
K = 16
_BIG_I32 = 2**30

_DN = (((1,), (1,)), ((), ()))


def _gmins(dist):
    parts = [jnp.min(dist[:, g * G:(g + 1) * G], axis=1)[None, :]
             for g in range(NGB)]
    return jnp.concatenate(parts, axis=0)


def _dist_body(qsq_ref, ksq_ref, q_ref, kv_ref, dist_ref, gmin_ref):
    j = pl.program_id(0)
    qb = q_ref[...]
    kvb = kv_ref[...]
    cross = lax.dot_general(qb, kvb, _DN, preferred_element_type=jnp.float32)
    q_sq = qsq_ref[0, :]
    k_sq = ksq_ref[0, 0, :]
    dist = (q_sq[:, None] - 2.0 * cross) + k_sq[None, :]
    dist_ref[...] = dist
    gmin_ref[...] = _gmins(dist)

    @pl.when(j == NB - 1)
    def _mask_tail():
        col = (NB - 1) * BN + lax.broadcasted_iota(jnp.int32, (Q, BN), 1)
        dmask = jnp.where(col < N, dist, jnp.inf)
        dist_ref[...] = dmask
        gmin_ref[...] = _gmins(dmask)


def _stage1(q, kv, q_sq2, k_sq3):
    return pl.pallas_call(
        _dist_body,
        grid=(NB,),
        in_specs=[
            pl.BlockSpec((1, Q), lambda j: (0, 0)),
            pl.BlockSpec((1, 1, BN), lambda j: (j, 0, 0)),
            pl.BlockSpec((Q, D), lambda j: (0, 0)),
            pl.BlockSpec((BN, D), lambda j: (j, 0)),
        ],
        out_specs=[
            pl.BlockSpec((Q, BN), lambda j: (0, j)),
            pl.BlockSpec((NGB, Q), lambda j: (j, 0)),
        ],
        out_shape=[
            jax.ShapeDtypeStruct((Q, NPAD), jnp.float32),
            jax.ShapeDtypeStruct((NGRP, Q), jnp.float32),
        ],
    )(q_sq2, k_sq3, q, kv)


def _group_topk_body(gmin_ref, gsel_ref):
    g = gmin_ref[...]
    rowid = lax.broadcasted_iota(jnp.int32, (NGRP, Q), 0)
    for t in range(NSEL):
        m = jnp.min(g, axis=0)
        cand = jnp.where(g == m[None, :], rowid, _BIG_I32)
        sel = jnp.min(cand, axis=0)
        g = jnp.where(rowid == sel[None, :], jnp.inf, g)
        gsel_ref[t, :] = sel


def _stage2(gmin):
    return pl.pallas_call(
        _group_topk_body,
        grid=(1,),
        in_specs=[pl.BlockSpec((NGRP, Q), lambda j: (0, 0))],
        out_specs=pl.BlockSpec((NSEL, Q), lambda j: (0, 0)),
        out_shape=jax.ShapeDtypeStruct((NSEL, Q), jnp.int32),
    )(gmin)


def _sc_gather(table, row_ids):
    info = plsc.get_sparse_core_info()
    nc, ns = info.num_cores, info.num_subcores
    nw = nc * ns
    b_total = Q * NSEL
    bpw = b_total // nw
    nch = bpw // 128
    idx3 = row_ids.reshape(nw, nch, 128)
    mesh = plsc.VectorSubcoreMesh(core_axis_name="c", subcore_axis_name="s")

    @functools.partial(
        pl.kernel, mesh=mesh,
        out_type=jax.ShapeDtypeStruct((b_total, G), jnp.float32),
        scratch_types=[
            pltpu.VMEM((nch, 128), jnp.int32),
            pltpu.VMEM((bpw, G), jnp.float32),
            pltpu.SemaphoreType.DMA,
        ],
    )
    def gather_kernel(table_hbm, idx_hbm, out_hbm, idx_v, rows_v, sem):
        wid = lax.axis_index("s") * nc + lax.axis_index("c")
        pltpu.sync_copy(idx_hbm.at[wid], idx_v)
        handles = [
            pltpu.async_copy(table_hbm.at[idx_v.at[c]],
                             rows_v.at[pl.ds(c * 128, 128)], sem)
            for c in range(nch)
        ]
        for h in handles:
            h.wait()
        pltpu.sync_copy(rows_v, out_hbm.at[pl.ds(wid * bpw, bpw)])

    return gather_kernel(table, idx3)


def _final_body(cand_ref, gidx_ref, bd_ref, bi_ref):
    vals = cand_ref[...]
    gidx = gidx_ref[...]
    for t in range(K):
        m = jnp.min(vals, axis=1)
        eq = vals == m[:, None]
        sel = jnp.min(jnp.where(eq, gidx, _BIG_I32), axis=1)
        vals = jnp.where(gidx == sel[:, None], jnp.inf, vals)
        bd_ref[:, t] = m
        bi_ref[:, t] = sel


def _stage4(cand, gidx):
    return pl.pallas_call(
        _final_body,
        grid=(1,),
        in_specs=[
            pl.BlockSpec((Q, NCAND), lambda j: (0, 0)),
            pl.BlockSpec((Q, NCAND), lambda j: (0, 0)),
        ],
        out_specs=[
            pl.BlockSpec((Q, K), lambda j: (0, 0)),
            pl.BlockSpec((Q, K), lambda j: (0, 0)),
        ],
        out_shape=[
            jax.ShapeDtypeStruct((Q, K), jnp.float32),
            jax.ShapeDtypeStruct((Q, K), jnp.int32),
        ],
    )(cand, gidx)


def kernel(out_vectors, in_vectors, k):
    q_sq = jnp.sum(out_vectors * out_vectors, axis=-1)
    k_sq = jnp.sum(in_vectors * in_vectors, axis=-1)
    q_sq2 = q_sq.reshape(1, Q)
    k_sq3 = jnp.pad(k_sq, (0, NPAD - N)).reshape(NB, 1, BN)

    dist, gmin = _stage1(out_vectors, in_vectors, q_sq2, k_sq3)

    gsel_t = _stage2(gmin)
    gsel = gsel_t.T

    row_ids = (jnp.arange(Q, dtype=jnp.int32)[:, None] * NGRP
               + gsel).reshape(-1)
    table = dist.reshape(Q * NGRP, G)

    gathered = _sc_gather(table, row_ids)
    cand = gathered.reshape(Q, NCAND)
    gidx = (gsel[:, :, None] * G
            + jnp.arange(G, dtype=jnp.int32)[None, None, :]).reshape(Q, NCAND)

    best_dists, topk_idx = _stage4(cand, gidx)
    return (best_dists, topk_idx)


# speedup vs baseline: 1.3087x; 1.3087x over previous
import functools

import jax
import jax.numpy as jnp
from jax import lax
from jax.experimental import pallas as pl
from jax.experimental.pallas import tpu as pltpu
from jax.experimental.pallas import tpu_sc as plsc

Q = 512
D = 768
N = 100000
BN = 2048
NB = (N + BN - 1) // BN
NPAD = NB * BN
G = 128
NGB = BN // G
NGRP = NB * NGB
NSEL = 24
NCAND = NSEL * G
K = 16
_BIG_I32 = 2**30

_DN = (((1,), (1,)), ((), ()))


def _gmins(dist):
    parts = [jnp.min(dist[:, g * G:(g + 1) * G], axis=1)[None, :]
             for g in range(NGB)]
    return jnp.concatenate(parts, axis=0)


def _dist_body(qsq_ref, ksq_ref, q_ref, kv_ref, dist_ref, gmin_ref):
    j = pl.program_id(0)
    qb = q_ref[...]
    kvb = kv_ref[...]
    cross = lax.dot_general(qb, kvb, _DN, preferred_element_type=jnp.float32)
    q_sq = qsq_ref[0, :]
    k_sq = ksq_ref[0, 0, :]
    dist = (q_sq[:, None] - 2.0 * cross) + k_sq[None, :]
    dist_ref[...] = jnp.transpose(
        dist.reshape(Q, NGB, G), (1, 0, 2)).reshape(NGB * Q, G)
    gmin_ref[...] = _gmins(dist)

    @pl.when(j == NB - 1)
    def _mask_tail():
        col = (NB - 1) * BN + lax.broadcasted_iota(jnp.int32, (Q, BN), 1)
        dmask = jnp.where(col < N, dist, jnp.inf)
        dist_ref[...] = jnp.transpose(
            dmask.reshape(Q, NGB, G), (1, 0, 2)).reshape(NGB * Q, G)
        gmin_ref[...] = _gmins(dmask)


def _stage1(q, kv, q_sq2, k_sq3):
    return pl.pallas_call(
        _dist_body,
        grid=(NB,),
        in_specs=[
            pl.BlockSpec((1, Q), lambda j: (0, 0)),
            pl.BlockSpec((1, 1, BN), lambda j: (j, 0, 0)),
            pl.BlockSpec((Q, D), lambda j: (0, 0)),
            pl.BlockSpec((BN, D), lambda j: (j, 0)),
        ],
        out_specs=[
            pl.BlockSpec((NGB * Q, G), lambda j: (j, 0)),
            pl.BlockSpec((NGB, Q), lambda j: (j, 0)),
        ],
        out_shape=[
            jax.ShapeDtypeStruct((NGRP * Q, G), jnp.float32),
            jax.ShapeDtypeStruct((NGRP, Q), jnp.float32),
        ],
    )(q_sq2, k_sq3, q, kv)


def _group_topk_body(gmin_ref, gsel_ref):
    g = gmin_ref[...]
    rowid = lax.broadcasted_iota(jnp.int32, (NGRP, Q), 0)
    for t in range(NSEL):
        m = jnp.min(g, axis=0)
        cand = jnp.where(g == m[None, :], rowid, _BIG_I32)
        sel = jnp.min(cand, axis=0)
        g = jnp.where(rowid == sel[None, :], jnp.inf, g)
        gsel_ref[t, :] = sel


def _stage2(gmin):
    return pl.pallas_call(
        _group_topk_body,
        grid=(1,),
        in_specs=[pl.BlockSpec((NGRP, Q), lambda j: (0, 0))],
        out_specs=pl.BlockSpec((NSEL, Q), lambda j: (0, 0)),
        out_shape=jax.ShapeDtypeStruct((NSEL, Q), jnp.int32),
    )(gmin)


def _sc_gather(table, row_ids):
    info = plsc.get_sparse_core_info()
    nc, ns = info.num_cores, info.num_subcores
    nw = nc * ns
    b_total = Q * NSEL
    bpw = b_total // nw
    nch = bpw // 128
    idx3 = row_ids.reshape(nw, nch, 128)
    mesh = plsc.VectorSubcoreMesh(core_axis_name="c", subcore_axis_name="s")

    @functools.partial(
        pl.kernel, mesh=mesh,
        out_type=jax.ShapeDtypeStruct((b_total, G), jnp.float32),
        scratch_types=[
            pltpu.VMEM((nch, 128), jnp.int32),
            pltpu.VMEM((bpw, G), jnp.float32),
            pltpu.SemaphoreType.DMA,
        ],
    )
    def gather_kernel(table_hbm, idx_hbm, out_hbm, idx_v, rows_v, sem):
        wid = lax.axis_index("s") * nc + lax.axis_index("c")
        pltpu.sync_copy(idx_hbm.at[wid], idx_v)
        handles = [
            pltpu.async_copy(table_hbm.at[idx_v.at[c]],
                             rows_v.at[pl.ds(c * 128, 128)], sem)
            for c in range(nch)
        ]
        for h in handles:
            h.wait()
        pltpu.sync_copy(rows_v, out_hbm.at[pl.ds(wid * bpw, bpw)])

    return gather_kernel(table, idx3)


def _final_body(cand_ref, gidx_ref, bd_ref, bi_ref):
    vals = cand_ref[...]
    gidx = gidx_ref[...]
    for t in range(K):
        m = jnp.min(vals, axis=1)
        eq = vals == m[:, None]
        sel = jnp.min(jnp.where(eq, gidx, _BIG_I32), axis=1)
        vals = jnp.where(gidx == sel[:, None], jnp.inf, vals)
        bd_ref[:, t] = m
        bi_ref[:, t] = sel


def _stage4(cand, gidx):
    return pl.pallas_call(
        _final_body,
        grid=(1,),
        in_specs=[
            pl.BlockSpec((Q, NCAND), lambda j: (0, 0)),
            pl.BlockSpec((Q, NCAND), lambda j: (0, 0)),
        ],
        out_specs=[
            pl.BlockSpec((Q, K), lambda j: (0, 0)),
            pl.BlockSpec((Q, K), lambda j: (0, 0)),
        ],
        out_shape=[
            jax.ShapeDtypeStruct((Q, K), jnp.float32),
            jax.ShapeDtypeStruct((Q, K), jnp.int32),
        ],
    )(cand, gidx)


def kernel(out_vectors, in_vectors, k):
    q_sq = jnp.sum(out_vectors * out_vectors, axis=-1)
    k_sq = jnp.sum(in_vectors * in_vectors, axis=-1)
    q_sq2 = q_sq.reshape(1, Q)
    k_sq3 = jnp.pad(k_sq, (0, NPAD - N)).reshape(NB, 1, BN)

    dist, gmin = _stage1(out_vectors, in_vectors, q_sq2, k_sq3)

    gsel_t = _stage2(gmin)
    gsel = gsel_t.T

    row_ids = (gsel * Q
               + jnp.arange(Q, dtype=jnp.int32)[:, None]).reshape(-1)

    gathered = _sc_gather(dist, row_ids)
    cand = gathered.reshape(Q, NCAND)
    gidx = (gsel[:, :, None] * G
            + jnp.arange(G, dtype=jnp.int32)[None, None, :]).reshape(Q, NCAND)

    best_dists, topk_idx = _stage4(cand, gidx)
    return (best_dists, topk_idx)


# speedup vs baseline: 1.3700x; 1.0468x over previous
import functools

import jax
import jax.numpy as jnp
from jax import lax
from jax.experimental import pallas as pl
from jax.experimental.pallas import tpu as pltpu
from jax.experimental.pallas import tpu_sc as plsc

Q = 512
D = 768
N = 100000
BN = 4096
NB = (N + BN - 1) // BN
NPAD = NB * BN
G = 128
NGB = BN // G
NGRP = NB * NGB
NSEL = 24
NCAND = NSEL * G
K = 16
_BIG_I32 = 2**30

_DN = (((1,), (1,)), ((), ()))


def _gmins(dist):
    parts = [jnp.min(dist[:, g * G:(g + 1) * G], axis=1)[None, :]
             for g in range(NGB)]
    return jnp.concatenate(parts, axis=0)


def _dist_body(qsq_ref, ksq_ref, q_ref, kv_ref, dist_ref, gmin_ref):
    j = pl.program_id(0)
    qb = q_ref[...]
    kvb = kv_ref[...]
    cross = lax.dot_general(qb, kvb, _DN, preferred_element_type=jnp.float32)
    q_sq = qsq_ref[0, :]
    k_sq = ksq_ref[0, 0, :]
    dist = (q_sq[:, None] - 2.0 * cross) + k_sq[None, :]
    dist_ref[...] = jnp.transpose(
        dist.reshape(Q, NGB, G), (1, 0, 2)).reshape(NGB * Q, G)
    gmin_ref[...] = _gmins(dist)

    @pl.when(j == NB - 1)
    def _mask_tail():
        col = (NB - 1) * BN + lax.broadcasted_iota(jnp.int32, (Q, BN), 1)
        dmask = jnp.where(col < N, dist, jnp.inf)
        dist_ref[...] = jnp.transpose(
            dmask.reshape(Q, NGB, G), (1, 0, 2)).reshape(NGB * Q, G)
        gmin_ref[...] = _gmins(dmask)


def _stage1(q, kv, q_sq2, k_sq3):
    return pl.pallas_call(
        _dist_body,
        grid=(NB,),
        in_specs=[
            pl.BlockSpec((1, Q), lambda j: (0, 0)),
            pl.BlockSpec((1, 1, BN), lambda j: (j, 0, 0)),
            pl.BlockSpec((Q, D), lambda j: (0, 0)),
            pl.BlockSpec((BN, D), lambda j: (j, 0)),
        ],
        out_specs=[
            pl.BlockSpec((NGB * Q, G), lambda j: (j, 0)),
            pl.BlockSpec((NGB, Q), lambda j: (j, 0)),
        ],
        out_shape=[
            jax.ShapeDtypeStruct((NGRP * Q, G), jnp.float32),
            jax.ShapeDtypeStruct((NGRP, Q), jnp.float32),
        ],
    )(q_sq2, k_sq3, q, kv)


def _group_topk_body(gmin_ref, gsel_ref):
    g = gmin_ref[...]
    rowid = lax.broadcasted_iota(jnp.int32, (NGRP, Q), 0)
    for t in range(NSEL):
        m = jnp.min(g, axis=0)
        cand = jnp.where(g == m[None, :], rowid, _BIG_I32)
        sel = jnp.min(cand, axis=0)
        g = jnp.where(rowid == sel[None, :], jnp.inf, g)
        gsel_ref[t, :] = sel


def _stage2(gmin):
    return pl.pallas_call(
        _group_topk_body,
        grid=(1,),
        in_specs=[pl.BlockSpec((NGRP, Q), lambda j: (0, 0))],
        out_specs=pl.BlockSpec((NSEL, Q), lambda j: (0, 0)),
        out_shape=jax.ShapeDtypeStruct((NSEL, Q), jnp.int32),
    )(gmin)


def _sc_gather(table, row_ids):
    info = plsc.get_sparse_core_info()
    nc, ns = info.num_cores, info.num_subcores
    nw = nc * ns
    b_total = Q * NSEL
    bpw = b_total // nw
    nch = bpw // 128
    idx3 = row_ids.reshape(nw, nch, 128)
    mesh = plsc.VectorSubcoreMesh(core_axis_name="c", subcore_axis_name="s")

    @functools.partial(
        pl.kernel, mesh=mesh,
        out_type=jax.ShapeDtypeStruct((b_total, G), jnp.float32),
        scratch_types=[
            pltpu.VMEM((nch, 128), jnp.int32),
            pltpu.VMEM((bpw, G), jnp.float32),
            pltpu.SemaphoreType.DMA,
        ],
    )
    def gather_kernel(table_hbm, idx_hbm, out_hbm, idx_v, rows_v, sem):
        wid = lax.axis_index("s") * nc + lax.axis_index("c")
        pltpu.sync_copy(idx_hbm.at[wid], idx_v)
        handles = [
            pltpu.async_copy(table_hbm.at[idx_v.at[c]],
                             rows_v.at[pl.ds(c * 128, 128)], sem)
            for c in range(nch)
        ]
        for h in handles:
            h.wait()
        pltpu.sync_copy(rows_v, out_hbm.at[pl.ds(wid * bpw, bpw)])

    return gather_kernel(table, idx3)


def _final_body(cand_ref, gidx_ref, bd_ref, bi_ref):
    vals = cand_ref[...]
    gidx = gidx_ref[...]
    for t in range(K):
        m = jnp.min(vals, axis=1)
        eq = vals == m[:, None]
        sel = jnp.min(jnp.where(eq, gidx, _BIG_I32), axis=1)
        vals = jnp.where(gidx == sel[:, None], jnp.inf, vals)
        bd_ref[:, t] = m
        bi_ref[:, t] = sel


def _stage4(cand, gidx):
    return pl.pallas_call(
        _final_body,
        grid=(1,),
        in_specs=[
            pl.BlockSpec((Q, NCAND), lambda j: (0, 0)),
            pl.BlockSpec((Q, NCAND), lambda j: (0, 0)),
        ],
        out_specs=[
            pl.BlockSpec((Q, K), lambda j: (0, 0)),
            pl.BlockSpec((Q, K), lambda j: (0, 0)),
        ],
        out_shape=[
            jax.ShapeDtypeStruct((Q, K), jnp.float32),
            jax.ShapeDtypeStruct((Q, K), jnp.int32),
        ],
    )(cand, gidx)


def kernel(out_vectors, in_vectors, k):
    q_sq = jnp.sum(out_vectors * out_vectors, axis=-1)
    k_sq = jnp.sum(in_vectors * in_vectors, axis=-1)
    q_sq2 = q_sq.reshape(1, Q)
    k_sq3 = jnp.pad(k_sq, (0, NPAD - N)).reshape(NB, 1, BN)

    dist, gmin = _stage1(out_vectors, in_vectors, q_sq2, k_sq3)

    gsel_t = _stage2(gmin)
    gsel = gsel_t.T

    row_ids = (gsel * Q
               + jnp.arange(Q, dtype=jnp.int32)[:, None]).reshape(-1)

    gathered = _sc_gather(dist, row_ids)
    cand = gathered.reshape(Q, NCAND)
    gidx = (gsel[:, :, None] * G
            + jnp.arange(G, dtype=jnp.int32)[None, None, :]).reshape(Q, NCAND)

    best_dists, topk_idx = _stage4(cand, gidx)
    return (best_dists, topk_idx)
